# baseline (device time: 287220 ns/iter reference)
import jax
import jax.numpy as jnp
from jax import lax
from jax.experimental import pallas as pl
from jax.experimental.pallas import tpu as pltpu

N_DEV = 4
SQ = 512
D = 1024
HQ = 8
DH = 128
QCHUNK = 512
SCALE = 0.08838834764831843


def _ag_body(x_ref, o_ref, send_sems, recv_sems):
    my = lax.axis_index("i")
    left = lax.rem(my + N_DEV - 1, N_DEV)
    right = lax.rem(my + 1, N_DEV)

    barrier = pltpu.get_barrier_semaphore()
    for nbr in (left, right):
        pl.semaphore_signal(barrier, inc=1, device_id=(nbr,),
                            device_id_type=pl.DeviceIdType.MESH)
    pl.semaphore_wait(barrier, 2)

    o_ref[my] = x_ref[...]
    for h in range(N_DEV - 1):
        src = lax.rem(my + N_DEV - h, N_DEV)
        rdma = pltpu.make_async_remote_copy(
            src_ref=o_ref.at[src],
            dst_ref=o_ref.at[src],
            send_sem=send_sems.at[h],
            recv_sem=recv_sems.at[h],
            device_id=(right,),
            device_id_type=pl.DeviceIdType.MESH,
        )
        rdma.start()
        rdma.wait()


def _all_gather(x):
    return pl.pallas_call(
        _ag_body,
        out_shape=jax.ShapeDtypeStruct((N_DEV, SQ, D), jnp.float32),
        in_specs=[pl.BlockSpec(memory_space=pltpu.VMEM)],
        out_specs=pl.BlockSpec(memory_space=pltpu.VMEM),
        scratch_shapes=[
            pltpu.SemaphoreType.DMA((N_DEV - 1,)),
            pltpu.SemaphoreType.DMA((N_DEV - 1,)),
        ],
        compiler_params=pltpu.CompilerParams(collective_id=0),
    )(x)


def _rs_body(p_ref, o_ref, recv_buf, send_buf, send_sems, recv_sems):
    my = lax.axis_index("i")
    left = lax.rem(my + N_DEV - 1, N_DEV)
    right = lax.rem(my + 1, N_DEV)

    barrier = pltpu.get_barrier_semaphore()
    for nbr in (left, right):
        pl.semaphore_signal(barrier, inc=1, device_id=(nbr,),
                            device_id_type=pl.DeviceIdType.MESH)
    pl.semaphore_wait(barrier, 2)

    for s in range(N_DEV - 1):
        chunk = lax.rem(my + 2 * N_DEV - 1 - s, N_DEV)
        if s == 0:
            src = p_ref.at[chunk]
        else:
            send_buf[s % 2] = p_ref[chunk] + recv_buf[s - 1]
            src = send_buf.at[s % 2]
        rdma = pltpu.make_async_remote_copy(
            src_ref=src,
            dst_ref=recv_buf.at[s],
            send_sem=send_sems.at[s],
            recv_sem=recv_sems.at[s],
            device_id=(right,),
            device_id_type=pl.DeviceIdType.MESH,
        )
        rdma.start()
        rdma.wait()

    o_ref[...] = p_ref[my] + recv_buf[N_DEV - 2]


def _reduce_scatter(p):
    return pl.pallas_call(
        _rs_body,
        out_shape=jax.ShapeDtypeStruct((SQ, D), jnp.float32),
        in_specs=[pl.BlockSpec(memory_space=pltpu.VMEM)],
        out_specs=pl.BlockSpec(memory_space=pltpu.VMEM),
        scratch_shapes=[
            pltpu.VMEM((N_DEV - 1, SQ, D), jnp.float32),
            pltpu.VMEM((2, SQ, D), jnp.float32),
            pltpu.SemaphoreType.DMA((N_DEV - 1,)),
            pltpu.SemaphoreType.DMA((N_DEV - 1,)),
        ],
        compiler_params=pltpu.CompilerParams(collective_id=1),
    )(p)


def _mm_body(a_ref, b_ref, o_ref):
    o_ref[...] = jnp.dot(a_ref[...], b_ref[...],
                         preferred_element_type=jnp.float32)


def _matmul(a, b):
    return pl.pallas_call(
        _mm_body,
        out_shape=jax.ShapeDtypeStruct((a.shape[0], b.shape[1]), jnp.float32),
        in_specs=[pl.BlockSpec(memory_space=pltpu.VMEM)] * 2,
        out_specs=pl.BlockSpec(memory_space=pltpu.VMEM),
    )(a, b)


def _attn_body(q_ref, k_ref, v_ref, o_ref):
    s = lax.dot_general(
        q_ref[...], k_ref[...], (((1,), (1,)), ((), ())),
        preferred_element_type=jnp.float32,
    ) * SCALE
    m = jnp.max(s, axis=-1, keepdims=True)
    p = jnp.exp(s - m)
    l = jnp.sum(p, axis=-1, keepdims=True)
    ctx = jnp.dot(p, v_ref[...], preferred_element_type=jnp.float32)
    o_ref[...] = ctx / l


def _attention(Q, K, V):
    S = Q.shape[0]
    return pl.pallas_call(
        _attn_body,
        grid=(HQ, S // QCHUNK),
        in_specs=[
            pl.BlockSpec((QCHUNK, DH), lambda h, qc: (qc, h)),
            pl.BlockSpec((S, DH), lambda h, qc: (0, h)),
            pl.BlockSpec((S, DH), lambda h, qc: (0, h)),
        ],
        out_specs=pl.BlockSpec((QCHUNK, DH), lambda h, qc: (qc, h)),
        out_shape=jax.ShapeDtypeStruct((S, HQ * DH), jnp.float32),
    )(Q, K, V)


def kernel(x, Wq, Wo, Wk, Wv):
    xs = x.reshape(SQ, D)
    xg = _all_gather(xs).reshape(N_DEV * SQ, D)
    Q = _matmul(xg, Wq)
    K = _matmul(xg, Wk)
    V = _matmul(xg, Wv)
    ctx = _attention(Q, K, V)
    partial = _matmul(ctx, Wo)
    out = _reduce_scatter(partial.reshape(N_DEV, SQ, D))
    return out.reshape(1, SQ, D)


# device time: 200853 ns/iter; 1.4300x vs baseline; 1.4300x over previous
import jax
import jax.numpy as jnp
from jax import lax
from jax.experimental import pallas as pl
from jax.experimental.pallas import tpu as pltpu

N_DEV = 4
SQ = 512
D = 1024
HQ = 8
DH = 128
SCALE = 0.08838834764831843

_MESH = pl.DeviceIdType.MESH


def _ring_ids():
    my = lax.axis_index("i")
    left = lax.rem(my + N_DEV - 1, N_DEV)
    right = lax.rem(my + 1, N_DEV)
    return my, left, right


def _neighbor_barrier(left, right):
    barrier = pltpu.get_barrier_semaphore()
    for nbr in (left, right):
        pl.semaphore_signal(barrier, inc=1, device_id=(nbr,),
                            device_id_type=_MESH)
    pl.semaphore_wait(barrier, 2)


def _ag_qkv_body(x_ref, wq_ref, wk_ref, wv_ref,
                 q_ref, k_ref, v_ref,
                 xg_ref, send_sems, recv_sems):
    my, left, right = _ring_ids()
    _neighbor_barrier(left, right)

    def qkv(c):
        xc = xg_ref[c]
        q_ref[pl.ds(c * SQ, SQ), :] = jnp.dot(
            xc, wq_ref[...], preferred_element_type=jnp.float32)
        k_ref[pl.ds(c * SQ, SQ), :] = jnp.dot(
            xc, wk_ref[...], preferred_element_type=jnp.float32)
        v_ref[pl.ds(c * SQ, SQ), :] = jnp.dot(
            xc, wv_ref[...], preferred_element_type=jnp.float32)

    xg_ref[my] = x_ref[...]
    rdmas = []
    r0 = pltpu.make_async_remote_copy(
        src_ref=xg_ref.at[my], dst_ref=xg_ref.at[my],
        send_sem=send_sems.at[0], recv_sem=recv_sems.at[0],
        device_id=(right,), device_id_type=_MESH)
    r0.start()
    rdmas.append(r0)
    qkv(my)
    for h in range(1, N_DEV):
        rdmas[h - 1].wait_recv()
        c = lax.rem(my + N_DEV - h, N_DEV)
        if h < N_DEV - 1:
            r = pltpu.make_async_remote_copy(
                src_ref=xg_ref.at[c], dst_ref=xg_ref.at[c],
                send_sem=send_sems.at[h], recv_sem=recv_sems.at[h],
                device_id=(right,), device_id_type=_MESH)
            r.start()
            rdmas.append(r)
        qkv(c)
    for r in rdmas:
        r.wait_send()


def _ag_qkv(xs, Wq, Wk, Wv):
    out = jax.ShapeDtypeStruct((N_DEV * SQ, D), jnp.float32)
    return pl.pallas_call(
        _ag_qkv_body,
        out_shape=(out, out, out),
        in_specs=[pl.BlockSpec(memory_space=pltpu.VMEM)] * 4,
        out_specs=(pl.BlockSpec(memory_space=pltpu.VMEM),) * 3,
        scratch_shapes=[
            pltpu.VMEM((N_DEV, SQ, D), jnp.float32),
            pltpu.SemaphoreType.DMA((N_DEV - 1,)),
            pltpu.SemaphoreType.DMA((N_DEV - 1,)),
        ],
        compiler_params=pltpu.CompilerParams(
            collective_id=0, vmem_limit_bytes=100 * 1024 * 1024),
    )(xs, Wq, Wk, Wv)


def _attn_rs_body(q_hbm, k_ref, v_ref, wo_ref, o_ref,
                  qc_ref, ctx_ref, send_buf, recv_buf,
                  copy_sem, send_sems, recv_sems):
    my, left, right = _ring_ids()
    _neighbor_barrier(left, right)

    def partial_chunk(c):
        cp = pltpu.make_async_copy(
            q_hbm.at[pl.ds(c * SQ, SQ), :], qc_ref, copy_sem)
        cp.start()
        cp.wait()
        for h in range(HQ):
            sl = slice(h * DH, (h + 1) * DH)
            s = lax.dot_general(
                qc_ref[:, sl], k_ref[:, sl], (((1,), (1,)), ((), ())),
                preferred_element_type=jnp.float32,
            ) * SCALE
            m = jnp.max(s, axis=-1, keepdims=True)
            p = jnp.exp(s - m)
            l = jnp.sum(p, axis=-1, keepdims=True)
            ctx_ref[:, sl] = jnp.dot(
                p, v_ref[:, sl], preferred_element_type=jnp.float32) / l
        return jnp.dot(ctx_ref[...], wo_ref[...],
                       preferred_element_type=jnp.float32)

    rdmas = []
    for s_i in range(N_DEV - 1):
        c = lax.rem(my + 2 * N_DEV - 1 - s_i, N_DEV)
        part = partial_chunk(c)
        slot = s_i % 2
        if s_i >= 2:
            rdmas[s_i - 2].wait_send()
        if s_i == 0:
            send_buf[slot] = part
        else:
            rdmas[s_i - 1].wait_recv()
            send_buf[slot] = part + recv_buf[s_i - 1]
        r = pltpu.make_async_remote_copy(
            src_ref=send_buf.at[slot], dst_ref=recv_buf.at[s_i],
            send_sem=send_sems.at[s_i], recv_sem=recv_sems.at[s_i],
            device_id=(right,), device_id_type=_MESH)
        r.start()
        rdmas.append(r)
    part_my = partial_chunk(my)
    rdmas[N_DEV - 2].wait_recv()
    o_ref[...] = part_my + recv_buf[N_DEV - 2]
    for r in rdmas[N_DEV - 3:]:
        r.wait_send()


def _attn_rs(Q, K, V, Wo):
    return pl.pallas_call(
        _attn_rs_body,
        out_shape=jax.ShapeDtypeStruct((SQ, D), jnp.float32),
        in_specs=[
            pl.BlockSpec(memory_space=pl.ANY),
            pl.BlockSpec(memory_space=pltpu.VMEM),
            pl.BlockSpec(memory_space=pltpu.VMEM),
            pl.BlockSpec(memory_space=pltpu.VMEM),
        ],
        out_specs=pl.BlockSpec(memory_space=pltpu.VMEM),
        scratch_shapes=[
            pltpu.VMEM((SQ, D), jnp.float32),
            pltpu.VMEM((SQ, D), jnp.float32),
            pltpu.VMEM((2, SQ, D), jnp.float32),
            pltpu.VMEM((N_DEV - 1, SQ, D), jnp.float32),
            pltpu.SemaphoreType.DMA,
            pltpu.SemaphoreType.DMA((N_DEV - 1,)),
            pltpu.SemaphoreType.DMA((N_DEV - 1,)),
        ],
        compiler_params=pltpu.CompilerParams(
            collective_id=1, vmem_limit_bytes=100 * 1024 * 1024),
    )(Q, K, V, Wo)


def _ag_body_v1(x_ref, o_ref, send_sems, recv_sems):
    my, left, right = _ring_ids()
    _neighbor_barrier(left, right)
    o_ref[my] = x_ref[...]
    for h in range(N_DEV - 1):
        src = lax.rem(my + N_DEV - h, N_DEV)
        rdma = pltpu.make_async_remote_copy(
            src_ref=o_ref.at[src], dst_ref=o_ref.at[src],
            send_sem=send_sems.at[h], recv_sem=recv_sems.at[h],
            device_id=(right,), device_id_type=_MESH)
        rdma.start()
        rdma.wait()


def _all_gather_v1(x):
    return pl.pallas_call(
        _ag_body_v1,
        out_shape=jax.ShapeDtypeStruct((N_DEV, SQ, D), jnp.float32),
        in_specs=[pl.BlockSpec(memory_space=pltpu.VMEM)],
        out_specs=pl.BlockSpec(memory_space=pltpu.VMEM),
        scratch_shapes=[
            pltpu.SemaphoreType.DMA((N_DEV - 1,)),
            pltpu.SemaphoreType.DMA((N_DEV - 1,)),
        ],
        compiler_params=pltpu.CompilerParams(collective_id=0),
    )(x)


def _rs_body_v1(p_ref, o_ref, recv_buf, send_buf, send_sems, recv_sems):
    my, left, right = _ring_ids()
    _neighbor_barrier(left, right)
    for s in range(N_DEV - 1):
        chunk = lax.rem(my + 2 * N_DEV - 1 - s, N_DEV)
        if s == 0:
            src = p_ref.at[chunk]
        else:
            send_buf[s % 2] = p_ref[chunk] + recv_buf[s - 1]
            src = send_buf.at[s % 2]
        rdma = pltpu.make_async_remote_copy(
            src_ref=src, dst_ref=recv_buf.at[s],
            send_sem=send_sems.at[s], recv_sem=recv_sems.at[s],
            device_id=(right,), device_id_type=_MESH)
        rdma.start()
        rdma.wait()
    o_ref[...] = p_ref[my] + recv_buf[N_DEV - 2]


def _reduce_scatter_v1(p):
    return pl.pallas_call(
        _rs_body_v1,
        out_shape=jax.ShapeDtypeStruct((SQ, D), jnp.float32),
        in_specs=[pl.BlockSpec(memory_space=pltpu.VMEM)],
        out_specs=pl.BlockSpec(memory_space=pltpu.VMEM),
        scratch_shapes=[
            pltpu.VMEM((N_DEV - 1, SQ, D), jnp.float32),
            pltpu.VMEM((2, SQ, D), jnp.float32),
            pltpu.SemaphoreType.DMA((N_DEV - 1,)),
            pltpu.SemaphoreType.DMA((N_DEV - 1,)),
        ],
        compiler_params=pltpu.CompilerParams(collective_id=1),
    )(p)


def _mm_body(a_ref, b_ref, o_ref):
    o_ref[...] = jnp.dot(a_ref[...], b_ref[...],
                         preferred_element_type=jnp.float32)


def _matmul(a, b):
    return pl.pallas_call(
        _mm_body,
        out_shape=jax.ShapeDtypeStruct((a.shape[0], b.shape[1]), jnp.float32),
        in_specs=[pl.BlockSpec(memory_space=pltpu.VMEM)] * 2,
        out_specs=pl.BlockSpec(memory_space=pltpu.VMEM),
    )(a, b)


def _attn_body_v1(q_ref, k_ref, v_ref, o_ref):
    s = lax.dot_general(
        q_ref[...], k_ref[...], (((1,), (1,)), ((), ())),
        preferred_element_type=jnp.float32,
    ) * SCALE
    m = jnp.max(s, axis=-1, keepdims=True)
    p = jnp.exp(s - m)
    l = jnp.sum(p, axis=-1, keepdims=True)
    ctx = jnp.dot(p, v_ref[...], preferred_element_type=jnp.float32)
    o_ref[...] = ctx / l


def _attention_v1(Q, K, V):
    S = Q.shape[0]
    return pl.pallas_call(
        _attn_body_v1,
        grid=(HQ, S // SQ),
        in_specs=[
            pl.BlockSpec((SQ, DH), lambda h, qc: (qc, h)),
            pl.BlockSpec((S, DH), lambda h, qc: (0, h)),
            pl.BlockSpec((S, DH), lambda h, qc: (0, h)),
        ],
        out_specs=pl.BlockSpec((SQ, DH), lambda h, qc: (qc, h)),
        out_shape=jax.ShapeDtypeStruct((S, HQ * DH), jnp.float32),
    )(Q, K, V)


USE_FUSED_A = True
USE_FUSED_B = True


def kernel(x, Wq, Wo, Wk, Wv):
    xs = x.reshape(SQ, D)
    if USE_FUSED_A:
        Q, K, V = _ag_qkv(xs, Wq, Wk, Wv)
    else:
        xg = _all_gather_v1(xs).reshape(N_DEV * SQ, D)
        Q = _matmul(xg, Wq)
        K = _matmul(xg, Wk)
        V = _matmul(xg, Wv)
    if USE_FUSED_B:
        out = _attn_rs(Q, K, V, Wo)
    else:
        ctx = _attention_v1(Q, K, V)
        partial = _matmul(ctx, Wo)
        out = _reduce_scatter_v1(partial.reshape(N_DEV, SQ, D))
    return out.reshape(1, SQ, D)


# device time: 141910 ns/iter; 2.0240x vs baseline; 1.4154x over previous
import jax
import jax.numpy as jnp
from jax import lax
from jax.experimental import pallas as pl
from jax.experimental.pallas import tpu as pltpu

N_DEV = 4
SQ = 512
D = 1024
HQ = 8
DH = 128
SCALE = 0.08838834764831843

_MESH = pl.DeviceIdType.MESH


def _ring_ids():
    my = lax.axis_index("i")
    left = lax.rem(my + N_DEV - 1, N_DEV)
    right = lax.rem(my + 1, N_DEV)
    return my, left, right


def _neighbor_barrier(left, right):
    barrier = pltpu.get_barrier_semaphore()
    for nbr in (left, right):
        pl.semaphore_signal(barrier, inc=1, device_id=(nbr,),
                            device_id_type=_MESH)
    pl.semaphore_wait(barrier, 2)


def _ag_qkv_body(x_ref, wq_ref, wk_ref, wv_ref,
                 q_ref, k_ref, v_ref,
                 xg_ref, send_sems, recv_sems):
    my, left, right = _ring_ids()
    _neighbor_barrier(left, right)
    odt = q_ref.dtype

    def qkv(c):
        xc = xg_ref[c]
        q_ref[pl.ds(c * SQ, SQ), :] = jnp.dot(
            xc, wq_ref[...], preferred_element_type=jnp.float32).astype(odt)
        k_ref[pl.ds(c * SQ, SQ), :] = jnp.dot(
            xc, wk_ref[...], preferred_element_type=jnp.float32).astype(odt)
        v_ref[pl.ds(c * SQ, SQ), :] = jnp.dot(
            xc, wv_ref[...], preferred_element_type=jnp.float32).astype(odt)

    xg_ref[my] = x_ref[...]
    rdmas = []
    r0 = pltpu.make_async_remote_copy(
        src_ref=xg_ref.at[my], dst_ref=xg_ref.at[my],
        send_sem=send_sems.at[0], recv_sem=recv_sems.at[0],
        device_id=(right,), device_id_type=_MESH)
    r0.start()
    rdmas.append(r0)
    qkv(my)
    for h in range(1, N_DEV):
        rdmas[h - 1].wait_recv()
        c = lax.rem(my + N_DEV - h, N_DEV)
        if h < N_DEV - 1:
            r = pltpu.make_async_remote_copy(
                src_ref=xg_ref.at[c], dst_ref=xg_ref.at[c],
                send_sem=send_sems.at[h], recv_sem=recv_sems.at[h],
                device_id=(right,), device_id_type=_MESH)
            r.start()
            rdmas.append(r)
        qkv(c)
    for r in rdmas:
        r.wait_send()


def _ag_qkv(xs, Wq, Wk, Wv):
    dt = xs.dtype
    out = jax.ShapeDtypeStruct((N_DEV * SQ, D), dt)
    return pl.pallas_call(
        _ag_qkv_body,
        out_shape=(out, out, out),
        in_specs=[pl.BlockSpec(memory_space=pltpu.VMEM)] * 4,
        out_specs=(pl.BlockSpec(memory_space=pltpu.VMEM),) * 3,
        scratch_shapes=[
            pltpu.VMEM((N_DEV, SQ, D), dt),
            pltpu.SemaphoreType.DMA((N_DEV - 1,)),
            pltpu.SemaphoreType.DMA((N_DEV - 1,)),
        ],
        compiler_params=pltpu.CompilerParams(
            collective_id=0, vmem_limit_bytes=100 * 1024 * 1024),
    )(xs, Wq, Wk, Wv)


def _attn_rs_body(q_hbm, k_ref, v_ref, wo_ref, o_ref,
                  qc_ref, ctx_ref, send_buf, recv_buf,
                  copy_sem, send_sems, recv_sems):
    my, left, right = _ring_ids()
    _neighbor_barrier(left, right)

    cdt = ctx_ref.dtype

    def partial_chunk(c):
        cp = pltpu.make_async_copy(
            q_hbm.at[pl.ds(c * SQ, SQ), :], qc_ref, copy_sem)
        cp.start()
        cp.wait()
        for h in range(HQ):
            sl = slice(h * DH, (h + 1) * DH)
            s = lax.dot_general(
                qc_ref[:, sl], k_ref[:, sl], (((1,), (1,)), ((), ())),
                preferred_element_type=jnp.float32,
            ) * SCALE
            m = jnp.max(s, axis=-1, keepdims=True)
            p = jnp.exp(s - m)
            l = jnp.sum(p, axis=-1, keepdims=True)
            ctx_ref[:, sl] = (jnp.dot(
                p.astype(cdt), v_ref[:, sl],
                preferred_element_type=jnp.float32) / l).astype(cdt)
        return jnp.dot(ctx_ref[...], wo_ref[...],
                       preferred_element_type=jnp.float32)

    rdmas = []
    for s_i in range(N_DEV - 1):
        c = lax.rem(my + 2 * N_DEV - 1 - s_i, N_DEV)
        part = partial_chunk(c)
        slot = s_i % 2
        if s_i >= 2:
            rdmas[s_i - 2].wait_send()
        if s_i == 0:
            send_buf[slot] = part.astype(cdt)
        else:
            rdmas[s_i - 1].wait_recv()
            send_buf[slot] = (
                part + recv_buf[s_i - 1].astype(jnp.float32)).astype(cdt)
        r = pltpu.make_async_remote_copy(
            src_ref=send_buf.at[slot], dst_ref=recv_buf.at[s_i],
            send_sem=send_sems.at[s_i], recv_sem=recv_sems.at[s_i],
            device_id=(right,), device_id_type=_MESH)
        r.start()
        rdmas.append(r)
    part_my = partial_chunk(my)
    rdmas[N_DEV - 2].wait_recv()
    o_ref[...] = part_my + recv_buf[N_DEV - 2].astype(jnp.float32)
    for r in rdmas[N_DEV - 3:]:
        r.wait_send()


def _attn_rs(Q, K, V, Wo):
    dt = Q.dtype
    return pl.pallas_call(
        _attn_rs_body,
        out_shape=jax.ShapeDtypeStruct((SQ, D), jnp.float32),
        in_specs=[
            pl.BlockSpec(memory_space=pl.ANY),
            pl.BlockSpec(memory_space=pltpu.VMEM),
            pl.BlockSpec(memory_space=pltpu.VMEM),
            pl.BlockSpec(memory_space=pltpu.VMEM),
        ],
        out_specs=pl.BlockSpec(memory_space=pltpu.VMEM),
        scratch_shapes=[
            pltpu.VMEM((SQ, D), dt),
            pltpu.VMEM((SQ, D), dt),
            pltpu.VMEM((2, SQ, D), dt),
            pltpu.VMEM((N_DEV - 1, SQ, D), dt),
            pltpu.SemaphoreType.DMA,
            pltpu.SemaphoreType.DMA((N_DEV - 1,)),
            pltpu.SemaphoreType.DMA((N_DEV - 1,)),
        ],
        compiler_params=pltpu.CompilerParams(
            collective_id=1, vmem_limit_bytes=100 * 1024 * 1024),
    )(Q, K, V, Wo)


def _ag_body_v1(x_ref, o_ref, send_sems, recv_sems):
    my, left, right = _ring_ids()
    _neighbor_barrier(left, right)
    o_ref[my] = x_ref[...]
    for h in range(N_DEV - 1):
        src = lax.rem(my + N_DEV - h, N_DEV)
        rdma = pltpu.make_async_remote_copy(
            src_ref=o_ref.at[src], dst_ref=o_ref.at[src],
            send_sem=send_sems.at[h], recv_sem=recv_sems.at[h],
            device_id=(right,), device_id_type=_MESH)
        rdma.start()
        rdma.wait()


def _all_gather_v1(x):
    return pl.pallas_call(
        _ag_body_v1,
        out_shape=jax.ShapeDtypeStruct((N_DEV, SQ, D), jnp.float32),
        in_specs=[pl.BlockSpec(memory_space=pltpu.VMEM)],
        out_specs=pl.BlockSpec(memory_space=pltpu.VMEM),
        scratch_shapes=[
            pltpu.SemaphoreType.DMA((N_DEV - 1,)),
            pltpu.SemaphoreType.DMA((N_DEV - 1,)),
        ],
        compiler_params=pltpu.CompilerParams(collective_id=0),
    )(x)


def _rs_body_v1(p_ref, o_ref, recv_buf, send_buf, send_sems, recv_sems):
    my, left, right = _ring_ids()
    _neighbor_barrier(left, right)
    for s in range(N_DEV - 1):
        chunk = lax.rem(my + 2 * N_DEV - 1 - s, N_DEV)
        if s == 0:
            src = p_ref.at[chunk]
        else:
            send_buf[s % 2] = p_ref[chunk] + recv_buf[s - 1]
            src = send_buf.at[s % 2]
        rdma = pltpu.make_async_remote_copy(
            src_ref=src, dst_ref=recv_buf.at[s],
            send_sem=send_sems.at[s], recv_sem=recv_sems.at[s],
            device_id=(right,), device_id_type=_MESH)
        rdma.start()
        rdma.wait()
    o_ref[...] = p_ref[my] + recv_buf[N_DEV - 2]


def _reduce_scatter_v1(p):
    return pl.pallas_call(
        _rs_body_v1,
        out_shape=jax.ShapeDtypeStruct((SQ, D), jnp.float32),
        in_specs=[pl.BlockSpec(memory_space=pltpu.VMEM)],
        out_specs=pl.BlockSpec(memory_space=pltpu.VMEM),
        scratch_shapes=[
            pltpu.VMEM((N_DEV - 1, SQ, D), jnp.float32),
            pltpu.VMEM((2, SQ, D), jnp.float32),
            pltpu.SemaphoreType.DMA((N_DEV - 1,)),
            pltpu.SemaphoreType.DMA((N_DEV - 1,)),
        ],
        compiler_params=pltpu.CompilerParams(collective_id=1),
    )(p)


def _mm_body(a_ref, b_ref, o_ref):
    o_ref[...] = jnp.dot(a_ref[...], b_ref[...],
                         preferred_element_type=jnp.float32)


def _matmul(a, b):
    return pl.pallas_call(
        _mm_body,
        out_shape=jax.ShapeDtypeStruct((a.shape[0], b.shape[1]), jnp.float32),
        in_specs=[pl.BlockSpec(memory_space=pltpu.VMEM)] * 2,
        out_specs=pl.BlockSpec(memory_space=pltpu.VMEM),
    )(a, b)


def _attn_body_v1(q_ref, k_ref, v_ref, o_ref):
    s = lax.dot_general(
        q_ref[...], k_ref[...], (((1,), (1,)), ((), ())),
        preferred_element_type=jnp.float32,
    ) * SCALE
    m = jnp.max(s, axis=-1, keepdims=True)
    p = jnp.exp(s - m)
    l = jnp.sum(p, axis=-1, keepdims=True)
    ctx = jnp.dot(p, v_ref[...], preferred_element_type=jnp.float32)
    o_ref[...] = ctx / l


def _attention_v1(Q, K, V):
    S = Q.shape[0]
    return pl.pallas_call(
        _attn_body_v1,
        grid=(HQ, S // SQ),
        in_specs=[
            pl.BlockSpec((SQ, DH), lambda h, qc: (qc, h)),
            pl.BlockSpec((S, DH), lambda h, qc: (0, h)),
            pl.BlockSpec((S, DH), lambda h, qc: (0, h)),
        ],
        out_specs=pl.BlockSpec((SQ, DH), lambda h, qc: (qc, h)),
        out_shape=jax.ShapeDtypeStruct((S, HQ * DH), jnp.float32),
    )(Q, K, V)


USE_FUSED_A = True
USE_FUSED_B = True
GEMM_DTYPE = jnp.bfloat16


def kernel(x, Wq, Wo, Wk, Wv):
    xs = x.reshape(SQ, D).astype(GEMM_DTYPE)
    Wq = Wq.astype(GEMM_DTYPE)
    Wk = Wk.astype(GEMM_DTYPE)
    Wv = Wv.astype(GEMM_DTYPE)
    Wo = Wo.astype(GEMM_DTYPE)
    if USE_FUSED_A:
        Q, K, V = _ag_qkv(xs, Wq, Wk, Wv)
    else:
        xg = _all_gather_v1(xs).reshape(N_DEV * SQ, D)
        Q = _matmul(xg, Wq)
        K = _matmul(xg, Wk)
        V = _matmul(xg, Wv)
    if USE_FUSED_B:
        out = _attn_rs(Q, K, V, Wo)
    else:
        ctx = _attention_v1(Q, K, V)
        partial = _matmul(ctx, Wo)
        out = _reduce_scatter_v1(partial.reshape(N_DEV, SQ, D))
    return out.reshape(1, SQ, D)


# device time: 124936 ns/iter; 2.2989x vs baseline; 1.1359x over previous
import jax
import jax.numpy as jnp
from jax import lax
from jax.experimental import pallas as pl
from jax.experimental.pallas import tpu as pltpu

N_DEV = 4
SQ = 512
D = 1024
HQ = 8
DH = 128
SCALE = 0.08838834764831843

_MESH = pl.DeviceIdType.MESH


def _ring_ids():
    my = lax.axis_index("i")
    left = lax.rem(my + N_DEV - 1, N_DEV)
    right = lax.rem(my + 1, N_DEV)
    return my, left, right


def _neighbor_barrier(left, right):
    barrier = pltpu.get_barrier_semaphore()
    for nbr in (left, right):
        pl.semaphore_signal(barrier, inc=1, device_id=(nbr,),
                            device_id_type=_MESH)
    pl.semaphore_wait(barrier, 2)


NSUB = 4
SUBR = SQ // NSUB


def _ag_qkv_body(x_ref, wq_ref, wk_ref, wv_ref,
                 q_ref, k_ref, v_ref,
                 xg_ref, wqb, wkb, wvb, send_sems, recv_sems):
    my, left, right = _ring_ids()
    _neighbor_barrier(left, right)
    odt = q_ref.dtype

    wqb[...] = wq_ref[...].astype(odt)
    wkb[...] = wk_ref[...].astype(odt)
    wvb[...] = wv_ref[...].astype(odt)

    def qkv(slot):
        xc = xg_ref[slot]
        rows = pl.ds(slot * SUBR, SUBR)
        q_ref[rows, :] = jnp.dot(
            xc, wqb[...], preferred_element_type=jnp.float32).astype(odt)
        k_ref[rows, :] = jnp.dot(
            xc, wkb[...], preferred_element_type=jnp.float32).astype(odt)
        v_ref[rows, :] = jnp.dot(
            xc, wvb[...], preferred_element_type=jnp.float32).astype(odt)

    def fwd(slot, h, s):
        r = pltpu.make_async_remote_copy(
            src_ref=xg_ref.at[slot], dst_ref=xg_ref.at[slot],
            send_sem=send_sems.at[h * NSUB + s],
            recv_sem=recv_sems.at[h * NSUB + s],
            device_id=(right,), device_id_type=_MESH)
        r.start()
        return r

    rdmas = [[None] * NSUB for _ in range(N_DEV - 1)]
    for s in range(NSUB):
        xg_ref[my * NSUB + s] = x_ref[pl.ds(s * SUBR, SUBR), :].astype(odt)
    for s in range(NSUB):
        rdmas[0][s] = fwd(my * NSUB + s, 0, s)
    for s in range(NSUB):
        qkv(my * NSUB + s)
    for h in range(1, N_DEV):
        c = lax.rem(my + N_DEV - h, N_DEV)
        for s in range(NSUB):
            rdmas[h - 1][s].wait_recv()
            slot = c * NSUB + s
            if h < N_DEV - 1:
                rdmas[h][s] = fwd(slot, h, s)
            qkv(slot)
    for hs in rdmas:
        for r in hs:
            r.wait_send()


def _ag_qkv(xs, Wq, Wk, Wv):
    dt = jnp.bfloat16
    out = jax.ShapeDtypeStruct((N_DEV * SQ, D), dt)
    n_sem = (N_DEV - 1) * NSUB
    return pl.pallas_call(
        _ag_qkv_body,
        out_shape=(out, out, out),
        in_specs=[pl.BlockSpec(memory_space=pltpu.VMEM)] * 4,
        out_specs=(pl.BlockSpec(memory_space=pltpu.VMEM),) * 3,
        scratch_shapes=[
            pltpu.VMEM((N_DEV * NSUB, SUBR, D), dt),
            pltpu.VMEM((D, D), dt),
            pltpu.VMEM((D, D), dt),
            pltpu.VMEM((D, D), dt),
            pltpu.SemaphoreType.DMA((n_sem,)),
            pltpu.SemaphoreType.DMA((n_sem,)),
        ],
        compiler_params=pltpu.CompilerParams(
            collective_id=0, vmem_limit_bytes=100 * 1024 * 1024),
    )(xs, Wq, Wk, Wv)


def _attn_rs_body(q_hbm, k_ref, v_ref, wo_ref, o_ref,
                  qc_ref, ctx_ref, wob, send_buf, recv_buf,
                  copy_sem, send_sems, recv_sems):
    my, left, right = _ring_ids()
    _neighbor_barrier(left, right)

    cdt = ctx_ref.dtype
    wob[...] = wo_ref[...].astype(cdt)

    def partial_chunk(c):
        cp = pltpu.make_async_copy(
            q_hbm.at[pl.ds(c * SQ, SQ), :], qc_ref, copy_sem)
        cp.start()
        cp.wait()
        for h in range(HQ):
            sl = slice(h * DH, (h + 1) * DH)
            s = lax.dot_general(
                qc_ref[:, sl], k_ref[:, sl], (((1,), (1,)), ((), ())),
                preferred_element_type=jnp.float32,
            ) * SCALE
            m = jnp.max(s, axis=-1, keepdims=True)
            p = jnp.exp(s - m)
            l = jnp.sum(p, axis=-1, keepdims=True)
            ctx_ref[:, sl] = (jnp.dot(
                p.astype(cdt), v_ref[:, sl],
                preferred_element_type=jnp.float32) / l).astype(cdt)
        return jnp.dot(ctx_ref[...], wob[...],
                       preferred_element_type=jnp.float32)

    rdmas = []
    for s_i in range(N_DEV - 1):
        c = lax.rem(my + 2 * N_DEV - 1 - s_i, N_DEV)
        part = partial_chunk(c)
        slot = s_i % 2
        if s_i >= 2:
            rdmas[s_i - 2].wait_send()
        if s_i == 0:
            send_buf[slot] = part.astype(cdt)
        else:
            rdmas[s_i - 1].wait_recv()
            send_buf[slot] = (
                part + recv_buf[s_i - 1].astype(jnp.float32)).astype(cdt)
        r = pltpu.make_async_remote_copy(
            src_ref=send_buf.at[slot], dst_ref=recv_buf.at[s_i],
            send_sem=send_sems.at[s_i], recv_sem=recv_sems.at[s_i],
            device_id=(right,), device_id_type=_MESH)
        r.start()
        rdmas.append(r)
    part_my = partial_chunk(my)
    rdmas[N_DEV - 2].wait_recv()
    o_ref[...] = part_my + recv_buf[N_DEV - 2].astype(jnp.float32)
    for r in rdmas[N_DEV - 3:]:
        r.wait_send()


def _attn_rs(Q, K, V, Wo):
    dt = jnp.bfloat16
    return pl.pallas_call(
        _attn_rs_body,
        out_shape=jax.ShapeDtypeStruct((SQ, D), jnp.float32),
        in_specs=[
            pl.BlockSpec(memory_space=pl.ANY),
            pl.BlockSpec(memory_space=pltpu.VMEM),
            pl.BlockSpec(memory_space=pltpu.VMEM),
            pl.BlockSpec(memory_space=pltpu.VMEM),
        ],
        out_specs=pl.BlockSpec(memory_space=pltpu.VMEM),
        scratch_shapes=[
            pltpu.VMEM((SQ, D), dt),
            pltpu.VMEM((SQ, D), dt),
            pltpu.VMEM((D, D), dt),
            pltpu.VMEM((2, SQ, D), dt),
            pltpu.VMEM((N_DEV - 1, SQ, D), dt),
            pltpu.SemaphoreType.DMA,
            pltpu.SemaphoreType.DMA((N_DEV - 1,)),
            pltpu.SemaphoreType.DMA((N_DEV - 1,)),
        ],
        compiler_params=pltpu.CompilerParams(
            collective_id=1, vmem_limit_bytes=100 * 1024 * 1024),
    )(Q, K, V, Wo)


def _ag_body_v1(x_ref, o_ref, send_sems, recv_sems):
    my, left, right = _ring_ids()
    _neighbor_barrier(left, right)
    o_ref[my] = x_ref[...]
    for h in range(N_DEV - 1):
        src = lax.rem(my + N_DEV - h, N_DEV)
        rdma = pltpu.make_async_remote_copy(
            src_ref=o_ref.at[src], dst_ref=o_ref.at[src],
            send_sem=send_sems.at[h], recv_sem=recv_sems.at[h],
            device_id=(right,), device_id_type=_MESH)
        rdma.start()
        rdma.wait()


def _all_gather_v1(x):
    return pl.pallas_call(
        _ag_body_v1,
        out_shape=jax.ShapeDtypeStruct((N_DEV, SQ, D), jnp.float32),
        in_specs=[pl.BlockSpec(memory_space=pltpu.VMEM)],
        out_specs=pl.BlockSpec(memory_space=pltpu.VMEM),
        scratch_shapes=[
            pltpu.SemaphoreType.DMA((N_DEV - 1,)),
            pltpu.SemaphoreType.DMA((N_DEV - 1,)),
        ],
        compiler_params=pltpu.CompilerParams(collective_id=0),
    )(x)


def _rs_body_v1(p_ref, o_ref, recv_buf, send_buf, send_sems, recv_sems):
    my, left, right = _ring_ids()
    _neighbor_barrier(left, right)
    for s in range(N_DEV - 1):
        chunk = lax.rem(my + 2 * N_DEV - 1 - s, N_DEV)
        if s == 0:
            src = p_ref.at[chunk]
        else:
            send_buf[s % 2] = p_ref[chunk] + recv_buf[s - 1]
            src = send_buf.at[s % 2]
        rdma = pltpu.make_async_remote_copy(
            src_ref=src, dst_ref=recv_buf.at[s],
            send_sem=send_sems.at[s], recv_sem=recv_sems.at[s],
            device_id=(right,), device_id_type=_MESH)
        rdma.start()
        rdma.wait()
    o_ref[...] = p_ref[my] + recv_buf[N_DEV - 2]


def _reduce_scatter_v1(p):
    return pl.pallas_call(
        _rs_body_v1,
        out_shape=jax.ShapeDtypeStruct((SQ, D), jnp.float32),
        in_specs=[pl.BlockSpec(memory_space=pltpu.VMEM)],
        out_specs=pl.BlockSpec(memory_space=pltpu.VMEM),
        scratch_shapes=[
            pltpu.VMEM((N_DEV - 1, SQ, D), jnp.float32),
            pltpu.VMEM((2, SQ, D), jnp.float32),
            pltpu.SemaphoreType.DMA((N_DEV - 1,)),
            pltpu.SemaphoreType.DMA((N_DEV - 1,)),
        ],
        compiler_params=pltpu.CompilerParams(collective_id=1),
    )(p)


def _mm_body(a_ref, b_ref, o_ref):
    o_ref[...] = jnp.dot(a_ref[...], b_ref[...],
                         preferred_element_type=jnp.float32)


def _matmul(a, b):
    return pl.pallas_call(
        _mm_body,
        out_shape=jax.ShapeDtypeStruct((a.shape[0], b.shape[1]), jnp.float32),
        in_specs=[pl.BlockSpec(memory_space=pltpu.VMEM)] * 2,
        out_specs=pl.BlockSpec(memory_space=pltpu.VMEM),
    )(a, b)


def _attn_body_v1(q_ref, k_ref, v_ref, o_ref):
    s = lax.dot_general(
        q_ref[...], k_ref[...], (((1,), (1,)), ((), ())),
        preferred_element_type=jnp.float32,
    ) * SCALE
    m = jnp.max(s, axis=-1, keepdims=True)
    p = jnp.exp(s - m)
    l = jnp.sum(p, axis=-1, keepdims=True)
    ctx = jnp.dot(p, v_ref[...], preferred_element_type=jnp.float32)
    o_ref[...] = ctx / l


def _attention_v1(Q, K, V):
    S = Q.shape[0]
    return pl.pallas_call(
        _attn_body_v1,
        grid=(HQ, S // SQ),
        in_specs=[
            pl.BlockSpec((SQ, DH), lambda h, qc: (qc, h)),
            pl.BlockSpec((S, DH), lambda h, qc: (0, h)),
            pl.BlockSpec((S, DH), lambda h, qc: (0, h)),
        ],
        out_specs=pl.BlockSpec((SQ, DH), lambda h, qc: (qc, h)),
        out_shape=jax.ShapeDtypeStruct((S, HQ * DH), jnp.float32),
    )(Q, K, V)


USE_FUSED_A = True
USE_FUSED_B = True


def kernel(x, Wq, Wo, Wk, Wv):
    xs = x.reshape(SQ, D)
    if USE_FUSED_A:
        Q, K, V = _ag_qkv(xs, Wq, Wk, Wv)
    else:
        xg = _all_gather_v1(xs).reshape(N_DEV * SQ, D)
        Q = _matmul(xg, Wq)
        K = _matmul(xg, Wk)
        V = _matmul(xg, Wv)
    if USE_FUSED_B:
        out = _attn_rs(Q, K, V, Wo)
    else:
        ctx = _attention_v1(Q, K, V)
        partial = _matmul(ctx, Wo)
        out = _reduce_scatter_v1(partial.reshape(N_DEV, SQ, D))
    return out.reshape(1, SQ, D)


# device time: 123211 ns/iter; 2.3311x vs baseline; 1.0140x over previous
import jax
import jax.numpy as jnp
from jax import lax
from jax.experimental import pallas as pl
from jax.experimental.pallas import tpu as pltpu

N_DEV = 4
SQ = 512
D = 1024
HQ = 8
DH = 128
SCALE = 0.08838834764831843

_MESH = pl.DeviceIdType.MESH


def _ring_ids():
    my = lax.axis_index("i")
    left = lax.rem(my + N_DEV - 1, N_DEV)
    right = lax.rem(my + 1, N_DEV)
    return my, left, right


def _neighbor_barrier(left, right):
    barrier = pltpu.get_barrier_semaphore()
    for nbr in (left, right):
        pl.semaphore_signal(barrier, inc=1, device_id=(nbr,),
                            device_id_type=_MESH)
    pl.semaphore_wait(barrier, 2)


NSUB = 4
SUBR = SQ // NSUB


def _ag_qkv_body(x_ref, wq_ref, wk_ref, wv_ref,
                 q_ref, k_ref, v_ref,
                 xg_ref, wqb, wkb, wvb, send_sems, recv_sems):
    my, left, right = _ring_ids()
    _neighbor_barrier(left, right)
    odt = q_ref.dtype

    wqb[...] = wq_ref[...].astype(odt)
    wkb[...] = wk_ref[...].astype(odt)
    wvb[...] = wv_ref[...].astype(odt)

    def qkv(slot):
        xc = xg_ref[slot]
        rows = pl.ds(slot * SUBR, SUBR)
        q_ref[rows, :] = jnp.dot(
            xc, wqb[...], preferred_element_type=jnp.float32).astype(odt)
        k_ref[rows, :] = jnp.dot(
            xc, wkb[...], preferred_element_type=jnp.float32).astype(odt)
        v_ref[rows, :] = jnp.dot(
            xc, wvb[...], preferred_element_type=jnp.float32).astype(odt)

    def fwd(slot, h, s):
        r = pltpu.make_async_remote_copy(
            src_ref=xg_ref.at[slot], dst_ref=xg_ref.at[slot],
            send_sem=send_sems.at[h * NSUB + s],
            recv_sem=recv_sems.at[h * NSUB + s],
            device_id=(right,), device_id_type=_MESH)
        r.start()
        return r

    rdmas = [[None] * NSUB for _ in range(N_DEV - 1)]
    for s in range(NSUB):
        xg_ref[my * NSUB + s] = x_ref[pl.ds(s * SUBR, SUBR), :].astype(odt)
    for s in range(NSUB):
        rdmas[0][s] = fwd(my * NSUB + s, 0, s)
    for s in range(NSUB):
        qkv(my * NSUB + s)
    for h in range(1, N_DEV):
        c = lax.rem(my + N_DEV - h, N_DEV)
        for s in range(NSUB):
            rdmas[h - 1][s].wait_recv()
            slot = c * NSUB + s
            if h < N_DEV - 1:
                rdmas[h][s] = fwd(slot, h, s)
            qkv(slot)
    for hs in rdmas:
        for r in hs:
            r.wait_send()


def _ag_qkv(xs, Wq, Wk, Wv):
    dt = jnp.bfloat16
    out = jax.ShapeDtypeStruct((N_DEV * SQ, D), dt)
    n_sem = (N_DEV - 1) * NSUB
    return pl.pallas_call(
        _ag_qkv_body,
        out_shape=(out, out, out),
        in_specs=[pl.BlockSpec(memory_space=pltpu.VMEM)] * 4,
        out_specs=(pl.BlockSpec(memory_space=pltpu.VMEM),) * 3,
        scratch_shapes=[
            pltpu.VMEM((N_DEV * NSUB, SUBR, D), dt),
            pltpu.VMEM((D, D), dt),
            pltpu.VMEM((D, D), dt),
            pltpu.VMEM((D, D), dt),
            pltpu.SemaphoreType.DMA((n_sem,)),
            pltpu.SemaphoreType.DMA((n_sem,)),
        ],
        compiler_params=pltpu.CompilerParams(
            collective_id=0, vmem_limit_bytes=100 * 1024 * 1024),
    )(xs, Wq, Wk, Wv)


def _attn_rs_body(q_hbm, k_ref, v_ref, wo_ref, o_ref,
                  qc_ref, ctx_ref, wob, send_buf, recv_buf,
                  copy_sem, send_sems, recv_sems):
    my, left, right = _ring_ids()
    _neighbor_barrier(left, right)

    cdt = ctx_ref.dtype
    wob[...] = wo_ref[...].astype(cdt)

    def partial_chunk(c):
        cp = pltpu.make_async_copy(
            q_hbm.at[pl.ds(c * SQ, SQ), :], qc_ref, copy_sem)
        cp.start()
        cp.wait()
        for h in range(HQ):
            sl = slice(h * DH, (h + 1) * DH)
            s = lax.dot_general(
                qc_ref[:, sl], k_ref[:, sl], (((1,), (1,)), ((), ())),
                preferred_element_type=jnp.float32,
            ) * SCALE
            m = jnp.max(s, axis=-1, keepdims=True)
            p = jnp.exp(s - m)
            l = jnp.sum(p, axis=-1, keepdims=True)
            ctx_ref[:, sl] = (jnp.dot(
                p.astype(cdt), v_ref[:, sl],
                preferred_element_type=jnp.float32) / l).astype(cdt)
        return jnp.dot(ctx_ref[...], wob[...],
                       preferred_element_type=jnp.float32)

    rdmas = []
    for s_i in range(N_DEV - 1):
        c = lax.rem(my + 2 * N_DEV - 1 - s_i, N_DEV)
        part = partial_chunk(c)
        slot = s_i % 2
        if s_i >= 2:
            rdmas[s_i - 2].wait_send()
        if s_i == 0:
            send_buf[slot] = part.astype(cdt)
        else:
            rdmas[s_i - 1].wait_recv()
            send_buf[slot] = (
                part + recv_buf[s_i - 1].astype(jnp.float32)).astype(cdt)
        r = pltpu.make_async_remote_copy(
            src_ref=send_buf.at[slot], dst_ref=recv_buf.at[s_i],
            send_sem=send_sems.at[s_i], recv_sem=recv_sems.at[s_i],
            device_id=(right,), device_id_type=_MESH)
        r.start()
        rdmas.append(r)
    part_my = partial_chunk(my)
    rdmas[N_DEV - 2].wait_recv()
    o_ref[...] = part_my + recv_buf[N_DEV - 2].astype(jnp.float32)
    for r in rdmas[N_DEV - 3:]:
        r.wait_send()


def _attn_rs(Q, K, V, Wo):
    dt = jnp.bfloat16
    return pl.pallas_call(
        _attn_rs_body,
        out_shape=jax.ShapeDtypeStruct((SQ, D), jnp.float32),
        in_specs=[
            pl.BlockSpec(memory_space=pl.ANY),
            pl.BlockSpec(memory_space=pltpu.VMEM),
            pl.BlockSpec(memory_space=pltpu.VMEM),
            pl.BlockSpec(memory_space=pltpu.VMEM),
        ],
        out_specs=pl.BlockSpec(memory_space=pltpu.VMEM),
        scratch_shapes=[
            pltpu.VMEM((SQ, D), dt),
            pltpu.VMEM((SQ, D), dt),
            pltpu.VMEM((D, D), dt),
            pltpu.VMEM((2, SQ, D), dt),
            pltpu.VMEM((N_DEV - 1, SQ, D), dt),
            pltpu.SemaphoreType.DMA,
            pltpu.SemaphoreType.DMA((N_DEV - 1,)),
            pltpu.SemaphoreType.DMA((N_DEV - 1,)),
        ],
        compiler_params=pltpu.CompilerParams(
            collective_id=1, vmem_limit_bytes=100 * 1024 * 1024),
    )(Q, K, V, Wo)


def _cast_w_body(wq_ref, wk_ref, wv_ref, wo_ref, qb, kb, vb, ob):
    qb[...] = wq_ref[...].astype(jnp.bfloat16)
    kb[...] = wk_ref[...].astype(jnp.bfloat16)
    vb[...] = wv_ref[...].astype(jnp.bfloat16)
    ob[...] = wo_ref[...].astype(jnp.bfloat16)


def _cast_w(Wq, Wk, Wv, Wo):
    o = jax.ShapeDtypeStruct((D, D), jnp.bfloat16)
    return pl.pallas_call(
        _cast_w_body,
        out_shape=(o, o, o, o),
        in_specs=[pl.BlockSpec(memory_space=pltpu.VMEM)] * 4,
        out_specs=(pl.BlockSpec(memory_space=pltpu.VMEM),) * 4,
    )(Wq, Wk, Wv, Wo)


def _mha_body(x_ref, wqb, wkb, wvb, wob, o_ref,
              xg_ref, q_ref, k_ref, v_ref, ctx_ref, send_buf, recv_buf,
              ag_ssem, ag_rsem, rs_ssem, rs_rsem):
    my, left, right = _ring_ids()
    _neighbor_barrier(left, right)
    dt = q_ref.dtype

    def qkv(slot):
        xc = xg_ref[slot]
        rows = pl.ds(slot * SUBR, SUBR)
        q_ref[rows, :] = jnp.dot(
            xc, wqb[...], preferred_element_type=jnp.float32).astype(dt)
        k_ref[rows, :] = jnp.dot(
            xc, wkb[...], preferred_element_type=jnp.float32).astype(dt)
        v_ref[rows, :] = jnp.dot(
            xc, wvb[...], preferred_element_type=jnp.float32).astype(dt)

    def fwd(slot, h, s):
        r = pltpu.make_async_remote_copy(
            src_ref=xg_ref.at[slot], dst_ref=xg_ref.at[slot],
            send_sem=ag_ssem.at[h * NSUB + s],
            recv_sem=ag_rsem.at[h * NSUB + s],
            device_id=(right,), device_id_type=_MESH)
        r.start()
        return r

    ag = [[None] * NSUB for _ in range(N_DEV - 1)]
    for s in range(NSUB):
        xg_ref[my * NSUB + s] = x_ref[pl.ds(s * SUBR, SUBR), :].astype(dt)
    for s in range(NSUB):
        ag[0][s] = fwd(my * NSUB + s, 0, s)
    for s in range(NSUB):
        qkv(my * NSUB + s)
    for h in range(1, N_DEV):
        c = lax.rem(my + N_DEV - h, N_DEV)
        for s in range(NSUB):
            ag[h - 1][s].wait_recv()
            slot = c * NSUB + s
            if h < N_DEV - 1:
                ag[h][s] = fwd(slot, h, s)
            qkv(slot)
    for hs in ag:
        for r in hs:
            r.wait_send()

    def partial_chunk(c):
        for h in range(HQ):
            sl = slice(h * DH, (h + 1) * DH)
            s = lax.dot_general(
                q_ref[pl.ds(c * SQ, SQ), sl], k_ref[:, sl],
                (((1,), (1,)), ((), ())),
                preferred_element_type=jnp.float32,
            ) * SCALE
            m = jnp.max(s, axis=-1, keepdims=True)
            p = jnp.exp(s - m)
            l = jnp.sum(p, axis=-1, keepdims=True)
            ctx_ref[:, sl] = (jnp.dot(
                p.astype(dt), v_ref[:, sl],
                preferred_element_type=jnp.float32) / l).astype(dt)
        return jnp.dot(ctx_ref[...], wob[...],
                       preferred_element_type=jnp.float32)

    rdmas = []
    for s_i in range(N_DEV - 1):
        c = lax.rem(my + 2 * N_DEV - 1 - s_i, N_DEV)
        part = partial_chunk(c)
        slot = s_i % 2
        if s_i >= 2:
            rdmas[s_i - 2].wait_send()
        if s_i == 0:
            send_buf[slot] = part.astype(dt)
        else:
            rdmas[s_i - 1].wait_recv()
            send_buf[slot] = (
                part + recv_buf[s_i - 1].astype(jnp.float32)).astype(dt)
        r = pltpu.make_async_remote_copy(
            src_ref=send_buf.at[slot], dst_ref=recv_buf.at[s_i],
            send_sem=rs_ssem.at[s_i], recv_sem=rs_rsem.at[s_i],
            device_id=(right,), device_id_type=_MESH)
        r.start()
        rdmas.append(r)
    part_my = partial_chunk(my)
    rdmas[N_DEV - 2].wait_recv()
    o_ref[...] = part_my + recv_buf[N_DEV - 2].astype(jnp.float32)
    for r in rdmas[N_DEV - 3:]:
        r.wait_send()


def _mha_fused(xs, Wq, Wk, Wv, Wo):
    dt = jnp.bfloat16
    wqb, wkb, wvb, wob = _cast_w(Wq, Wk, Wv, Wo)
    n_ag = (N_DEV - 1) * NSUB
    return pl.pallas_call(
        _mha_body,
        out_shape=jax.ShapeDtypeStruct((SQ, D), jnp.float32),
        in_specs=[pl.BlockSpec(memory_space=pltpu.VMEM)] * 5,
        out_specs=pl.BlockSpec(memory_space=pltpu.VMEM),
        scratch_shapes=[
            pltpu.VMEM((N_DEV * NSUB, SUBR, D), dt),
            pltpu.VMEM((N_DEV * SQ, D), dt),
            pltpu.VMEM((N_DEV * SQ, D), dt),
            pltpu.VMEM((N_DEV * SQ, D), dt),
            pltpu.VMEM((SQ, D), dt),
            pltpu.VMEM((2, SQ, D), dt),
            pltpu.VMEM((N_DEV - 1, SQ, D), dt),
            pltpu.SemaphoreType.DMA((n_ag,)),
            pltpu.SemaphoreType.DMA((n_ag,)),
            pltpu.SemaphoreType.DMA((N_DEV - 1,)),
            pltpu.SemaphoreType.DMA((N_DEV - 1,)),
        ],
        compiler_params=pltpu.CompilerParams(
            collective_id=0, vmem_limit_bytes=100 * 1024 * 1024),
    )(xs, wqb, wkb, wvb, wob)


def _ag_body_v1(x_ref, o_ref, send_sems, recv_sems):
    my, left, right = _ring_ids()
    _neighbor_barrier(left, right)
    o_ref[my] = x_ref[...]
    for h in range(N_DEV - 1):
        src = lax.rem(my + N_DEV - h, N_DEV)
        rdma = pltpu.make_async_remote_copy(
            src_ref=o_ref.at[src], dst_ref=o_ref.at[src],
            send_sem=send_sems.at[h], recv_sem=recv_sems.at[h],
            device_id=(right,), device_id_type=_MESH)
        rdma.start()
        rdma.wait()


def _all_gather_v1(x):
    return pl.pallas_call(
        _ag_body_v1,
        out_shape=jax.ShapeDtypeStruct((N_DEV, SQ, D), jnp.float32),
        in_specs=[pl.BlockSpec(memory_space=pltpu.VMEM)],
        out_specs=pl.BlockSpec(memory_space=pltpu.VMEM),
        scratch_shapes=[
            pltpu.SemaphoreType.DMA((N_DEV - 1,)),
            pltpu.SemaphoreType.DMA((N_DEV - 1,)),
        ],
        compiler_params=pltpu.CompilerParams(collective_id=0),
    )(x)


def _rs_body_v1(p_ref, o_ref, recv_buf, send_buf, send_sems, recv_sems):
    my, left, right = _ring_ids()
    _neighbor_barrier(left, right)
    for s in range(N_DEV - 1):
        chunk = lax.rem(my + 2 * N_DEV - 1 - s, N_DEV)
        if s == 0:
            src = p_ref.at[chunk]
        else:
            send_buf[s % 2] = p_ref[chunk] + recv_buf[s - 1]
            src = send_buf.at[s % 2]
        rdma = pltpu.make_async_remote_copy(
            src_ref=src, dst_ref=recv_buf.at[s],
            send_sem=send_sems.at[s], recv_sem=recv_sems.at[s],
            device_id=(right,), device_id_type=_MESH)
        rdma.start()
        rdma.wait()
    o_ref[...] = p_ref[my] + recv_buf[N_DEV - 2]


def _reduce_scatter_v1(p):
    return pl.pallas_call(
        _rs_body_v1,
        out_shape=jax.ShapeDtypeStruct((SQ, D), jnp.float32),
        in_specs=[pl.BlockSpec(memory_space=pltpu.VMEM)],
        out_specs=pl.BlockSpec(memory_space=pltpu.VMEM),
        scratch_shapes=[
            pltpu.VMEM((N_DEV - 1, SQ, D), jnp.float32),
            pltpu.VMEM((2, SQ, D), jnp.float32),
            pltpu.SemaphoreType.DMA((N_DEV - 1,)),
            pltpu.SemaphoreType.DMA((N_DEV - 1,)),
        ],
        compiler_params=pltpu.CompilerParams(collective_id=1),
    )(p)


def _mm_body(a_ref, b_ref, o_ref):
    o_ref[...] = jnp.dot(a_ref[...], b_ref[...],
                         preferred_element_type=jnp.float32)


def _matmul(a, b):
    return pl.pallas_call(
        _mm_body,
        out_shape=jax.ShapeDtypeStruct((a.shape[0], b.shape[1]), jnp.float32),
        in_specs=[pl.BlockSpec(memory_space=pltpu.VMEM)] * 2,
        out_specs=pl.BlockSpec(memory_space=pltpu.VMEM),
    )(a, b)


def _attn_body_v1(q_ref, k_ref, v_ref, o_ref):
    s = lax.dot_general(
        q_ref[...], k_ref[...], (((1,), (1,)), ((), ())),
        preferred_element_type=jnp.float32,
    ) * SCALE
    m = jnp.max(s, axis=-1, keepdims=True)
    p = jnp.exp(s - m)
    l = jnp.sum(p, axis=-1, keepdims=True)
    ctx = jnp.dot(p, v_ref[...], preferred_element_type=jnp.float32)
    o_ref[...] = ctx / l


def _attention_v1(Q, K, V):
    S = Q.shape[0]
    return pl.pallas_call(
        _attn_body_v1,
        grid=(HQ, S // SQ),
        in_specs=[
            pl.BlockSpec((SQ, DH), lambda h, qc: (qc, h)),
            pl.BlockSpec((S, DH), lambda h, qc: (0, h)),
            pl.BlockSpec((S, DH), lambda h, qc: (0, h)),
        ],
        out_specs=pl.BlockSpec((SQ, DH), lambda h, qc: (qc, h)),
        out_shape=jax.ShapeDtypeStruct((S, HQ * DH), jnp.float32),
    )(Q, K, V)


USE_FUSED_A = True
USE_FUSED_B = True
FUSE_ALL = True


def kernel(x, Wq, Wo, Wk, Wv):
    xs = x.reshape(SQ, D)
    if FUSE_ALL:
        return _mha_fused(xs, Wq, Wk, Wv, Wo).reshape(1, SQ, D)
    if USE_FUSED_A:
        Q, K, V = _ag_qkv(xs, Wq, Wk, Wv)
    else:
        xg = _all_gather_v1(xs).reshape(N_DEV * SQ, D)
        Q = _matmul(xg, Wq)
        K = _matmul(xg, Wk)
        V = _matmul(xg, Wv)
    if USE_FUSED_B:
        out = _attn_rs(Q, K, V, Wo)
    else:
        ctx = _attention_v1(Q, K, V)
        partial = _matmul(ctx, Wo)
        out = _reduce_scatter_v1(partial.reshape(N_DEV, SQ, D))
    return out.reshape(1, SQ, D)


# device time: 120132 ns/iter; 2.3909x vs baseline; 1.0256x over previous
import jax
import jax.numpy as jnp
from jax import lax
from jax.experimental import pallas as pl
from jax.experimental.pallas import tpu as pltpu

N_DEV = 4
SQ = 512
D = 1024
HQ = 8
DH = 128
SCALE = 0.08838834764831843

_MESH = pl.DeviceIdType.MESH


def _ring_ids():
    my = lax.axis_index("i")
    left = lax.rem(my + N_DEV - 1, N_DEV)
    right = lax.rem(my + 1, N_DEV)
    return my, left, right


def _neighbor_barrier(left, right):
    barrier = pltpu.get_barrier_semaphore()
    for nbr in (left, right):
        pl.semaphore_signal(barrier, inc=1, device_id=(nbr,),
                            device_id_type=_MESH)
    pl.semaphore_wait(barrier, 2)


NSUB = 4
SUBR = SQ // NSUB


def _ag_qkv_body(x_ref, wq_ref, wk_ref, wv_ref,
                 q_ref, k_ref, v_ref,
                 xg_ref, wqb, wkb, wvb, send_sems, recv_sems):
    my, left, right = _ring_ids()
    _neighbor_barrier(left, right)
    odt = q_ref.dtype

    wqb[...] = wq_ref[...].astype(odt)
    wkb[...] = wk_ref[...].astype(odt)
    wvb[...] = wv_ref[...].astype(odt)

    def qkv(slot):
        xc = xg_ref[slot]
        rows = pl.ds(slot * SUBR, SUBR)
        q_ref[rows, :] = jnp.dot(
            xc, wqb[...], preferred_element_type=jnp.float32).astype(odt)
        k_ref[rows, :] = jnp.dot(
            xc, wkb[...], preferred_element_type=jnp.float32).astype(odt)
        v_ref[rows, :] = jnp.dot(
            xc, wvb[...], preferred_element_type=jnp.float32).astype(odt)

    def fwd(slot, h, s):
        r = pltpu.make_async_remote_copy(
            src_ref=xg_ref.at[slot], dst_ref=xg_ref.at[slot],
            send_sem=send_sems.at[h * NSUB + s],
            recv_sem=recv_sems.at[h * NSUB + s],
            device_id=(right,), device_id_type=_MESH)
        r.start()
        return r

    rdmas = [[None] * NSUB for _ in range(N_DEV - 1)]
    for s in range(NSUB):
        xg_ref[my * NSUB + s] = x_ref[pl.ds(s * SUBR, SUBR), :].astype(odt)
    for s in range(NSUB):
        rdmas[0][s] = fwd(my * NSUB + s, 0, s)
    for s in range(NSUB):
        qkv(my * NSUB + s)
    for h in range(1, N_DEV):
        c = lax.rem(my + N_DEV - h, N_DEV)
        for s in range(NSUB):
            rdmas[h - 1][s].wait_recv()
            slot = c * NSUB + s
            if h < N_DEV - 1:
                rdmas[h][s] = fwd(slot, h, s)
            qkv(slot)
    for hs in rdmas:
        for r in hs:
            r.wait_send()


def _ag_qkv(xs, Wq, Wk, Wv):
    dt = jnp.bfloat16
    out = jax.ShapeDtypeStruct((N_DEV * SQ, D), dt)
    n_sem = (N_DEV - 1) * NSUB
    return pl.pallas_call(
        _ag_qkv_body,
        out_shape=(out, out, out),
        in_specs=[pl.BlockSpec(memory_space=pltpu.VMEM)] * 4,
        out_specs=(pl.BlockSpec(memory_space=pltpu.VMEM),) * 3,
        scratch_shapes=[
            pltpu.VMEM((N_DEV * NSUB, SUBR, D), dt),
            pltpu.VMEM((D, D), dt),
            pltpu.VMEM((D, D), dt),
            pltpu.VMEM((D, D), dt),
            pltpu.SemaphoreType.DMA((n_sem,)),
            pltpu.SemaphoreType.DMA((n_sem,)),
        ],
        compiler_params=pltpu.CompilerParams(
            collective_id=0, vmem_limit_bytes=100 * 1024 * 1024),
    )(xs, Wq, Wk, Wv)


def _attn_rs_body(q_hbm, k_ref, v_ref, wo_ref, o_ref,
                  qc_ref, ctx_ref, wob, send_buf, recv_buf,
                  copy_sem, send_sems, recv_sems):
    my, left, right = _ring_ids()
    _neighbor_barrier(left, right)

    cdt = ctx_ref.dtype
    wob[...] = wo_ref[...].astype(cdt)

    def partial_chunk(c):
        cp = pltpu.make_async_copy(
            q_hbm.at[pl.ds(c * SQ, SQ), :], qc_ref, copy_sem)
        cp.start()
        cp.wait()
        for h in range(HQ):
            sl = slice(h * DH, (h + 1) * DH)
            s = lax.dot_general(
                qc_ref[:, sl], k_ref[:, sl], (((1,), (1,)), ((), ())),
                preferred_element_type=jnp.float32,
            ) * SCALE
            m = jnp.max(s, axis=-1, keepdims=True)
            p = jnp.exp(s - m)
            l = jnp.sum(p, axis=-1, keepdims=True)
            ctx_ref[:, sl] = (jnp.dot(
                p.astype(cdt), v_ref[:, sl],
                preferred_element_type=jnp.float32) / l).astype(cdt)
        return jnp.dot(ctx_ref[...], wob[...],
                       preferred_element_type=jnp.float32)

    rdmas = []
    for s_i in range(N_DEV - 1):
        c = lax.rem(my + 2 * N_DEV - 1 - s_i, N_DEV)
        part = partial_chunk(c)
        slot = s_i % 2
        if s_i >= 2:
            rdmas[s_i - 2].wait_send()
        if s_i == 0:
            send_buf[slot] = part.astype(cdt)
        else:
            rdmas[s_i - 1].wait_recv()
            send_buf[slot] = (
                part + recv_buf[s_i - 1].astype(jnp.float32)).astype(cdt)
        r = pltpu.make_async_remote_copy(
            src_ref=send_buf.at[slot], dst_ref=recv_buf.at[s_i],
            send_sem=send_sems.at[s_i], recv_sem=recv_sems.at[s_i],
            device_id=(right,), device_id_type=_MESH)
        r.start()
        rdmas.append(r)
    part_my = partial_chunk(my)
    rdmas[N_DEV - 2].wait_recv()
    o_ref[...] = part_my + recv_buf[N_DEV - 2].astype(jnp.float32)
    for r in rdmas[N_DEV - 3:]:
        r.wait_send()


def _attn_rs(Q, K, V, Wo):
    dt = jnp.bfloat16
    return pl.pallas_call(
        _attn_rs_body,
        out_shape=jax.ShapeDtypeStruct((SQ, D), jnp.float32),
        in_specs=[
            pl.BlockSpec(memory_space=pl.ANY),
            pl.BlockSpec(memory_space=pltpu.VMEM),
            pl.BlockSpec(memory_space=pltpu.VMEM),
            pl.BlockSpec(memory_space=pltpu.VMEM),
        ],
        out_specs=pl.BlockSpec(memory_space=pltpu.VMEM),
        scratch_shapes=[
            pltpu.VMEM((SQ, D), dt),
            pltpu.VMEM((SQ, D), dt),
            pltpu.VMEM((D, D), dt),
            pltpu.VMEM((2, SQ, D), dt),
            pltpu.VMEM((N_DEV - 1, SQ, D), dt),
            pltpu.SemaphoreType.DMA,
            pltpu.SemaphoreType.DMA((N_DEV - 1,)),
            pltpu.SemaphoreType.DMA((N_DEV - 1,)),
        ],
        compiler_params=pltpu.CompilerParams(
            collective_id=1, vmem_limit_bytes=100 * 1024 * 1024),
    )(Q, K, V, Wo)


def _cast_w_body(wq_ref, wk_ref, wv_ref, wo_ref, qb, kb, vb, ob):
    qb[...] = wq_ref[...].astype(jnp.bfloat16)
    kb[...] = wk_ref[...].astype(jnp.bfloat16)
    vb[...] = wv_ref[...].astype(jnp.bfloat16)
    ob[...] = wo_ref[...].astype(jnp.bfloat16)


def _cast_w(Wq, Wk, Wv, Wo):
    o = jax.ShapeDtypeStruct((D, D), jnp.bfloat16)
    return pl.pallas_call(
        _cast_w_body,
        out_shape=(o, o, o, o),
        in_specs=[pl.BlockSpec(memory_space=pltpu.VMEM)] * 4,
        out_specs=(pl.BlockSpec(memory_space=pltpu.VMEM),) * 4,
    )(Wq, Wk, Wv, Wo)


def _mha_body(x_ref, wqb, wkb, wvb, wob, o_ref,
              xg_ref, q_ref, k_ref, v_ref, ctx_ref, send_buf, recv_buf,
              ag_ssem, ag_rsem, rs_ssem, rs_rsem):
    my, left, right = _ring_ids()
    _neighbor_barrier(left, right)
    dt = q_ref.dtype

    def qkv(slot):
        xc = xg_ref[slot]
        rows = pl.ds(slot * SUBR, SUBR)
        q_ref[rows, :] = (jnp.dot(
            xc, wqb[...], preferred_element_type=jnp.float32)
            * SCALE).astype(dt)
        k_ref[rows, :] = jnp.dot(
            xc, wkb[...], preferred_element_type=jnp.float32).astype(dt)
        v_ref[rows, :] = jnp.dot(
            xc, wvb[...], preferred_element_type=jnp.float32).astype(dt)

    def fwd(slot, h, s):
        r = pltpu.make_async_remote_copy(
            src_ref=xg_ref.at[slot], dst_ref=xg_ref.at[slot],
            send_sem=ag_ssem.at[h * NSUB + s],
            recv_sem=ag_rsem.at[h * NSUB + s],
            device_id=(right,), device_id_type=_MESH)
        r.start()
        return r

    ag = [[None] * NSUB for _ in range(N_DEV - 1)]
    for s in range(NSUB):
        xg_ref[my * NSUB + s] = x_ref[pl.ds(s * SUBR, SUBR), :].astype(dt)
    for s in range(NSUB):
        ag[0][s] = fwd(my * NSUB + s, 0, s)
    for s in range(NSUB):
        qkv(my * NSUB + s)
    for h in range(1, N_DEV):
        c = lax.rem(my + N_DEV - h, N_DEV)
        for s in range(NSUB):
            ag[h - 1][s].wait_recv()
            slot = c * NSUB + s
            if h < N_DEV - 1:
                ag[h][s] = fwd(slot, h, s)
            qkv(slot)
    for hs in ag:
        for r in hs:
            r.wait_send()

    def partial_chunk(c):
        for h in range(HQ):
            sl = slice(h * DH, (h + 1) * DH)
            s = lax.dot_general(
                q_ref[pl.ds(c * SQ, SQ), sl], k_ref[:, sl],
                (((1,), (1,)), ((), ())),
                preferred_element_type=jnp.float32,
            )
            m = jnp.max(s, axis=-1, keepdims=True)
            p = jnp.exp(s - m)
            l = jnp.sum(p, axis=-1, keepdims=True)
            ctx_ref[:, sl] = (jnp.dot(
                p.astype(dt), v_ref[:, sl],
                preferred_element_type=jnp.float32) / l).astype(dt)
        return jnp.dot(ctx_ref[...], wob[...],
                       preferred_element_type=jnp.float32)

    rdmas = []
    for s_i in range(N_DEV - 1):
        c = lax.rem(my + 2 * N_DEV - 1 - s_i, N_DEV)
        part = partial_chunk(c)
        slot = s_i % 2
        if s_i >= 2:
            rdmas[s_i - 2].wait_send()
        if s_i == 0:
            send_buf[slot] = part.astype(dt)
        else:
            rdmas[s_i - 1].wait_recv()
            send_buf[slot] = (
                part + recv_buf[s_i - 1].astype(jnp.float32)).astype(dt)
        r = pltpu.make_async_remote_copy(
            src_ref=send_buf.at[slot], dst_ref=recv_buf.at[s_i],
            send_sem=rs_ssem.at[s_i], recv_sem=rs_rsem.at[s_i],
            device_id=(right,), device_id_type=_MESH)
        r.start()
        rdmas.append(r)
    part_my = partial_chunk(my)
    rdmas[N_DEV - 2].wait_recv()
    o_ref[...] = part_my + recv_buf[N_DEV - 2].astype(jnp.float32)
    for r in rdmas[N_DEV - 3:]:
        r.wait_send()


def _mha_fused(xs, Wq, Wk, Wv, Wo):
    dt = jnp.bfloat16
    wqb, wkb, wvb, wob = _cast_w(Wq, Wk, Wv, Wo)
    n_ag = (N_DEV - 1) * NSUB
    return pl.pallas_call(
        _mha_body,
        out_shape=jax.ShapeDtypeStruct((SQ, D), jnp.float32),
        in_specs=[pl.BlockSpec(memory_space=pltpu.VMEM)] * 5,
        out_specs=pl.BlockSpec(memory_space=pltpu.VMEM),
        scratch_shapes=[
            pltpu.VMEM((N_DEV * NSUB, SUBR, D), dt),
            pltpu.VMEM((N_DEV * SQ, D), dt),
            pltpu.VMEM((N_DEV * SQ, D), dt),
            pltpu.VMEM((N_DEV * SQ, D), dt),
            pltpu.VMEM((SQ, D), dt),
            pltpu.VMEM((2, SQ, D), dt),
            pltpu.VMEM((N_DEV - 1, SQ, D), dt),
            pltpu.SemaphoreType.DMA((n_ag,)),
            pltpu.SemaphoreType.DMA((n_ag,)),
            pltpu.SemaphoreType.DMA((N_DEV - 1,)),
            pltpu.SemaphoreType.DMA((N_DEV - 1,)),
        ],
        compiler_params=pltpu.CompilerParams(
            collective_id=0, vmem_limit_bytes=100 * 1024 * 1024),
    )(xs, wqb, wkb, wvb, wob)


def _ag_body_v1(x_ref, o_ref, send_sems, recv_sems):
    my, left, right = _ring_ids()
    _neighbor_barrier(left, right)
    o_ref[my] = x_ref[...]
    for h in range(N_DEV - 1):
        src = lax.rem(my + N_DEV - h, N_DEV)
        rdma = pltpu.make_async_remote_copy(
            src_ref=o_ref.at[src], dst_ref=o_ref.at[src],
            send_sem=send_sems.at[h], recv_sem=recv_sems.at[h],
            device_id=(right,), device_id_type=_MESH)
        rdma.start()
        rdma.wait()


def _all_gather_v1(x):
    return pl.pallas_call(
        _ag_body_v1,
        out_shape=jax.ShapeDtypeStruct((N_DEV, SQ, D), jnp.float32),
        in_specs=[pl.BlockSpec(memory_space=pltpu.VMEM)],
        out_specs=pl.BlockSpec(memory_space=pltpu.VMEM),
        scratch_shapes=[
            pltpu.SemaphoreType.DMA((N_DEV - 1,)),
            pltpu.SemaphoreType.DMA((N_DEV - 1,)),
        ],
        compiler_params=pltpu.CompilerParams(collective_id=0),
    )(x)


def _rs_body_v1(p_ref, o_ref, recv_buf, send_buf, send_sems, recv_sems):
    my, left, right = _ring_ids()
    _neighbor_barrier(left, right)
    for s in range(N_DEV - 1):
        chunk = lax.rem(my + 2 * N_DEV - 1 - s, N_DEV)
        if s == 0:
            src = p_ref.at[chunk]
        else:
            send_buf[s % 2] = p_ref[chunk] + recv_buf[s - 1]
            src = send_buf.at[s % 2]
        rdma = pltpu.make_async_remote_copy(
            src_ref=src, dst_ref=recv_buf.at[s],
            send_sem=send_sems.at[s], recv_sem=recv_sems.at[s],
            device_id=(right,), device_id_type=_MESH)
        rdma.start()
        rdma.wait()
    o_ref[...] = p_ref[my] + recv_buf[N_DEV - 2]


def _reduce_scatter_v1(p):
    return pl.pallas_call(
        _rs_body_v1,
        out_shape=jax.ShapeDtypeStruct((SQ, D), jnp.float32),
        in_specs=[pl.BlockSpec(memory_space=pltpu.VMEM)],
        out_specs=pl.BlockSpec(memory_space=pltpu.VMEM),
        scratch_shapes=[
            pltpu.VMEM((N_DEV - 1, SQ, D), jnp.float32),
            pltpu.VMEM((2, SQ, D), jnp.float32),
            pltpu.SemaphoreType.DMA((N_DEV - 1,)),
            pltpu.SemaphoreType.DMA((N_DEV - 1,)),
        ],
        compiler_params=pltpu.CompilerParams(collective_id=1),
    )(p)


def _mm_body(a_ref, b_ref, o_ref):
    o_ref[...] = jnp.dot(a_ref[...], b_ref[...],
                         preferred_element_type=jnp.float32)


def _matmul(a, b):
    return pl.pallas_call(
        _mm_body,
        out_shape=jax.ShapeDtypeStruct((a.shape[0], b.shape[1]), jnp.float32),
        in_specs=[pl.BlockSpec(memory_space=pltpu.VMEM)] * 2,
        out_specs=pl.BlockSpec(memory_space=pltpu.VMEM),
    )(a, b)


def _attn_body_v1(q_ref, k_ref, v_ref, o_ref):
    s = lax.dot_general(
        q_ref[...], k_ref[...], (((1,), (1,)), ((), ())),
        preferred_element_type=jnp.float32,
    ) * SCALE
    m = jnp.max(s, axis=-1, keepdims=True)
    p = jnp.exp(s - m)
    l = jnp.sum(p, axis=-1, keepdims=True)
    ctx = jnp.dot(p, v_ref[...], preferred_element_type=jnp.float32)
    o_ref[...] = ctx / l


def _attention_v1(Q, K, V):
    S = Q.shape[0]
    return pl.pallas_call(
        _attn_body_v1,
        grid=(HQ, S // SQ),
        in_specs=[
            pl.BlockSpec((SQ, DH), lambda h, qc: (qc, h)),
            pl.BlockSpec((S, DH), lambda h, qc: (0, h)),
            pl.BlockSpec((S, DH), lambda h, qc: (0, h)),
        ],
        out_specs=pl.BlockSpec((SQ, DH), lambda h, qc: (qc, h)),
        out_shape=jax.ShapeDtypeStruct((S, HQ * DH), jnp.float32),
    )(Q, K, V)


USE_FUSED_A = True
USE_FUSED_B = True
FUSE_ALL = True


def kernel(x, Wq, Wo, Wk, Wv):
    xs = x.reshape(SQ, D)
    if FUSE_ALL:
        return _mha_fused(xs, Wq, Wk, Wv, Wo).reshape(1, SQ, D)
    if USE_FUSED_A:
        Q, K, V = _ag_qkv(xs, Wq, Wk, Wv)
    else:
        xg = _all_gather_v1(xs).reshape(N_DEV * SQ, D)
        Q = _matmul(xg, Wq)
        K = _matmul(xg, Wk)
        V = _matmul(xg, Wv)
    if USE_FUSED_B:
        out = _attn_rs(Q, K, V, Wo)
    else:
        ctx = _attention_v1(Q, K, V)
        partial = _matmul(ctx, Wo)
        out = _reduce_scatter_v1(partial.reshape(N_DEV, SQ, D))
    return out.reshape(1, SQ, D)
